# baseline (speedup 1.0000x reference)
import jax
import jax.numpy as jnp
from jax.experimental import pallas as pl

CONV = 8
EPS = 1e-5
N = 100000
G = 64


def _head_body(pooled_ref, Wh_ref, bh_ref, Wo_ref, bo_ref, out_ref):
    p = pooled_ref[...]
    h = jnp.maximum(jnp.dot(p, Wh_ref[...], preferred_element_type=jnp.float32) + bh_ref[...], 0.0)
    out_ref[...] = jnp.dot(h, Wo_ref[...], preferred_element_type=jnp.float32) + bo_ref[...]


def kernel(x, edge_index, batch, emb, W1, b1, W2, b2, Wh, bh, Wo, bo):
    h = jnp.take(emb, x, axis=0)
    src = edge_index[0]
    dst = edge_index[1]
    xs = [h]
    for i in range(CONV):
        msgs = jnp.take(h, src, axis=0)
        aggr = jax.ops.segment_sum(msgs, dst, num_segments=N)
        z = h + aggr
        z = jnp.maximum(jnp.dot(z, W1[i]) + b1[i], 0.0)
        z = jnp.dot(z, W2[i]) + b2[i]
        mean = jnp.mean(z, axis=0, keepdims=True)
        var = jnp.var(z, axis=0, keepdims=True)
        z = (z - mean) / jnp.sqrt(var + EPS)
        h = jax.nn.relu(h + z)
        xs.append(h)
    cat = jnp.concatenate(xs, axis=1)
    pooled = jax.ops.segment_max(cat, batch, num_segments=G)
    pooled = jnp.where(jnp.isfinite(pooled), pooled, 0.0)
    out = pl.pallas_call(
        _head_body,
        out_shape=jax.ShapeDtypeStruct((G, Wo.shape[1]), jnp.float32),
    )(pooled, Wh, bh.reshape(1, -1), Wo, bo.reshape(1, -1))
    return out


# speedup vs baseline: 9.1875x; 9.1875x over previous
import functools

import jax
import jax.numpy as jnp
import numpy as np
from jax import lax
from jax.experimental import pallas as pl
from jax.experimental.pallas import tpu as pltpu
from jax.experimental.pallas import tpu_sc as plsc

N = 100000
E = 1600000
C = 32
HC = 16
CONV = 8
GH = 64
H = 64
OUT = 32
G = 64
EPS = 1e-5

RW = 128
NROWP = 12544
RPT = NROWP // 16
R = 8
NWIN = RPT // R
NSP = 6256
NSL = N - 15 * NSP
ZC = 368
CHK = 3128
CHL = NSL - CHK

BN = 1000
NB = N // BN

PSP = 6256
PLAST = 6160
PCK = 512
PFULL = 12
PT0 = PSP - PFULL * PCK
PT1 = PLAST - PFULL * PCK
LAYERS = CONV + 1
PC = LAYERS * HC


def _segsum_body(h2, srcaug, dst2, aggr, idx_s, idx_d, rows, zbuf, acc,
                 sem_g, sem_s):
    c = lax.axis_index("c")
    s = lax.axis_index("s")
    t0 = s * NSP

    def zloop(i, carry):
        zbuf[i, :] = jnp.zeros((HC,), jnp.float32)
        return carry

    lax.fori_loop(0, ZC, zloop, 0)
    for k in range(16):
        pltpu.sync_copy(zbuf, acc.at[pl.ds(t0 + k * ZC, ZC)])

    @pl.when(s < 15)
    def _z1():
        pltpu.sync_copy(zbuf, acc.at[pl.ds(t0 + 16 * ZC, ZC)])

    @pl.when(s == 15)
    def _z2():
        pltpu.sync_copy(zbuf.at[pl.ds(0, NSL - 16 * ZC)],
                        acc.at[pl.ds(t0 + 16 * ZC, NSL - 16 * ZC)])
        pltpu.sync_copy(zbuf.at[pl.ds(0, 8)], acc.at[pl.ds(N, 8)])

    plsc.subcore_barrier()

    base0 = s * RPT

    def window(w, carry):
        r0 = base0 + w * R
        pltpu.sync_copy(srcaug.at[pl.ds(c * NROWP + r0, R)], idx_s)
        pltpu.sync_copy(dst2.at[pl.ds(r0, R)], idx_d)
        cps = [pltpu.async_copy(h2.at[idx_s.at[j]],
                                rows.at[pl.ds(j * RW, RW)], sem_g)
               for j in range(R)]
        for cp in cps:
            cp.wait()
        cps2 = [pltpu.async_copy(rows.at[pl.ds(j * RW, RW)],
                                 acc.at[idx_d.at[j]], sem_s, add=True)
                for j in range(R)]
        for cp in cps2:
            cp.wait()
        return carry

    lax.fori_loop(0, NWIN, window, 0)

    plsc.subcore_barrier()
    pltpu.sync_copy(acc.at[pl.ds(t0, CHK)], aggr.at[pl.ds(c * N + t0, CHK)])

    @pl.when(s < 15)
    def _w1():
        pltpu.sync_copy(acc.at[pl.ds(t0 + CHK, CHK)],
                        aggr.at[pl.ds(c * N + t0 + CHK, CHK)])

    @pl.when(s == 15)
    def _w2():
        pltpu.sync_copy(acc.at[pl.ds(t0 + CHK, CHL)],
                        aggr.at[pl.ds(c * N + t0 + CHK, CHL)])


_segsum = pl.kernel(
    _segsum_body,
    out_type=jax.ShapeDtypeStruct((2 * N, HC), jnp.float32),
    mesh=plsc.VectorSubcoreMesh(core_axis_name="c", subcore_axis_name="s"),
    scratch_types=[
        pltpu.VMEM((R, RW), jnp.int32),
        pltpu.VMEM((R, RW), jnp.int32),
        pltpu.VMEM((R * RW, HC), jnp.float32),
        pltpu.VMEM((ZC, HC), jnp.float32),
        pltpu.VMEM_SHARED((N + 8, HC), jnp.float32),
        pltpu.SemaphoreType.DMA,
        pltpu.SemaphoreType.DMA,
    ],
    compiler_params=pltpu.CompilerParams(use_tc_tiling_on_sc=False),
)


def _pool_body(h0, h1, h2, h3, h4, h5, h6, h7, h8, batch, out,
               bbuf, acc, bufs, sem):
    hs = (h0, h1, h2, h3, h4, h5, h6, h7, h8)
    c = lax.axis_index("c")
    s = lax.axis_index("s")
    t0 = s * PSP

    def init_g(g, carry):
        for k in range(LAYERS):
            acc[g, pl.ds(k * HC, HC)] = jnp.full((HC,), -jnp.inf,
                                                 jnp.float32)
        return carry

    lax.fori_loop(0, G, init_g, 0)

    @pl.when(s < 15)
    def _():
        pltpu.sync_copy(batch.at[pl.ds(t0, PSP)], bbuf)

    @pl.when(s == 15)
    def _():
        pltpu.sync_copy(batch.at[pl.ds(t0, PLAST)], bbuf.at[pl.ds(0, PLAST)])

    def do_rows(off, nrows):
        def rloop(r16, carry):
            bvec = bbuf[pl.ds(off + r16 * 16, 16)]
            for l in range(16):
                b = bvec[l]
                r = r16 * 16 + l
                for k in range(LAYERS):
                    sl = acc[b, pl.ds(k * HC, HC)]
                    acc[b, pl.ds(k * HC, HC)] = jnp.maximum(sl, bufs[k][r, :])
            return carry
        lax.fori_loop(0, nrows // 16, rloop, 0)

    def load_chunk(off, nrows):
        cps = [pltpu.async_copy(hs[k].at[pl.ds(c * N + t0 + off, nrows)],
                                bufs[k].at[pl.ds(0, nrows)], sem)
               for k in range(LAYERS)]
        for cp in cps:
            cp.wait()

    for ch in range(PFULL):
        load_chunk(ch * PCK, PCK)
        do_rows(ch * PCK, PCK)

    @pl.when(s < 15)
    def _tail0():
        load_chunk(PFULL * PCK, PT0)
        do_rows(PFULL * PCK, PT0)

    @pl.when(s == 15)
    def _tail1():
        load_chunk(PFULL * PCK, PT1)
        do_rows(PFULL * PCK, PT1)

    wid = c * 16 + s
    pltpu.sync_copy(acc, out.at[wid])


_pool = pl.kernel(
    _pool_body,
    out_type=jax.ShapeDtypeStruct((32, G, PC), jnp.float32),
    mesh=plsc.VectorSubcoreMesh(core_axis_name="c", subcore_axis_name="s"),
    scratch_types=[
        pltpu.VMEM((PSP,), jnp.int32),
        pltpu.VMEM((G, PC), jnp.float32),
        [pltpu.VMEM((PCK, HC), jnp.float32) for _ in range(LAYERS)],
        pltpu.SemaphoreType.DMA,
    ],
    compiler_params=pltpu.CompilerParams(use_tc_tiling_on_sc=False),
)


def _embed_body(x_ref, emb_ref, out_ref):
    xcol = x_ref[0]
    oh = (xcol == lax.broadcasted_iota(jnp.int32, (1, 8), 1))
    h = jnp.dot(oh.astype(jnp.float32), emb_ref[...],
                preferred_element_type=jnp.float32)
    out_ref[0] = h[:, :HC]
    out_ref[1] = h[:, HC:]


def _passA_body(h_ref, a_ref, W1_ref, b1_ref, W2_ref, b2_ref,
                z2_ref, st_ref):
    z = jnp.concatenate([h_ref[0] + a_ref[0], h_ref[1] + a_ref[1]], axis=1)
    z1 = jnp.maximum(
        jnp.dot(z, W1_ref[...], preferred_element_type=jnp.float32)
        + b1_ref[...], 0.0)
    z2 = (jnp.dot(z1, W2_ref[...], preferred_element_type=jnp.float32)
          + b2_ref[...])
    z2_ref[...] = z2
    st_ref[0, 0, :C] = jnp.sum(z2, axis=0)
    st_ref[0, 0, C:] = jnp.sum(z2 * z2, axis=0)


def _passB_body(st_ref, z2_ref, h_ref, out_ref):
    sums = jnp.sum(st_ref[...], axis=(0, 1)).reshape(1, 2 * C)
    mean = sums[:, :C] * (1.0 / N)
    var = sums[:, C:] * (1.0 / N) - mean * mean
    scale = lax.rsqrt(var + EPS)
    zn = (z2_ref[...] - mean) * scale
    out_ref[0] = jnp.maximum(h_ref[0] + zn[:, :HC], 0.0)
    out_ref[1] = jnp.maximum(h_ref[1] + zn[:, HC:], 0.0)


def _head_body(p0_ref, p1_ref, Wh_ref, bh_ref, Wo_ref, bo_ref, out_ref):
    m0 = jnp.max(p0_ref[...], axis=0)
    m1 = jnp.max(p1_ref[...], axis=0)
    pooled = jnp.concatenate([m0, m1], axis=1)
    pooled = jnp.where(jnp.isfinite(pooled), pooled, 0.0)
    hh = jnp.maximum(
        jnp.dot(pooled, Wh_ref[...], preferred_element_type=jnp.float32)
        + bh_ref[...], 0.0)
    out_ref[...] = (jnp.dot(hh, Wo_ref[...],
                            preferred_element_type=jnp.float32)
                    + bo_ref[...])


_embed = pl.pallas_call(
    _embed_body,
    grid=(NB,),
    in_specs=[
        pl.BlockSpec((1, BN, 1), lambda i: (i, 0, 0)),
        pl.BlockSpec((8, C), lambda i: (0, 0)),
    ],
    out_specs=pl.BlockSpec((2, BN, HC), lambda i: (0, i, 0)),
    out_shape=jax.ShapeDtypeStruct((2, N, HC), jnp.float32),
)

_passA = pl.pallas_call(
    _passA_body,
    grid=(NB,),
    in_specs=[
        pl.BlockSpec((2, BN, HC), lambda i: (0, i, 0)),
        pl.BlockSpec((2, BN, HC), lambda i: (0, i, 0)),
        pl.BlockSpec((C, GH), lambda i: (0, 0)),
        pl.BlockSpec((1, GH), lambda i: (0, 0)),
        pl.BlockSpec((GH, C), lambda i: (0, 0)),
        pl.BlockSpec((1, C), lambda i: (0, 0)),
    ],
    out_specs=[
        pl.BlockSpec((BN, C), lambda i: (i, 0)),
        pl.BlockSpec((1, 1, 2 * C), lambda i: (i, 0, 0)),
    ],
    out_shape=[
        jax.ShapeDtypeStruct((N, C), jnp.float32),
        jax.ShapeDtypeStruct((NB, 1, 2 * C), jnp.float32),
    ],
)

_passB = pl.pallas_call(
    _passB_body,
    grid=(NB,),
    in_specs=[
        pl.BlockSpec((NB, 1, 2 * C), lambda i: (0, 0, 0)),
        pl.BlockSpec((BN, C), lambda i: (i, 0)),
        pl.BlockSpec((2, BN, HC), lambda i: (0, i, 0)),
    ],
    out_specs=pl.BlockSpec((2, BN, HC), lambda i: (0, i, 0)),
    out_shape=jax.ShapeDtypeStruct((2, N, HC), jnp.float32),
)

_head = pl.pallas_call(
    _head_body,
    out_shape=jax.ShapeDtypeStruct((G, OUT), jnp.float32),
)

_PERM = np.array([k * C + c * HC + j
                  for c in range(2) for k in range(LAYERS)
                  for j in range(HC)])


def kernel(x, edge_index, batch, emb, W1, b1, W2, b2, Wh, bh, Wo, bo):
    src = edge_index[0]
    dst = edge_index[1]
    npad = NROWP * RW - E
    pad_i = jnp.arange(npad, dtype=jnp.int32)
    srcp = jnp.concatenate([src, (pad_i * 8) % N])
    dstp = jnp.concatenate([dst, N + (pad_i % 8)])
    srcaug = jnp.concatenate([srcp, srcp + N]).reshape(2 * NROWP, RW)
    dst2 = dstp.reshape(NROWP, RW)
    x3 = x.reshape(NB, BN, 1)
    embp = jnp.zeros((8, C), jnp.float32).at[:6].set(emb)

    h = _embed(x3, embp)
    hs = [h]
    for i in range(CONV):
        aggr2 = _segsum(h.reshape(2 * N, HC), srcaug, dst2)
        z2, stats = _passA(h, aggr2.reshape(2, N, HC), W1[i],
                           b1[i].reshape(1, GH), W2[i], b2[i].reshape(1, C))
        h = _passB(stats, z2, h)
        hs.append(h)

    pp = _pool(*[hh.reshape(2 * N, HC) for hh in hs], batch)
    pp = pp.reshape(2, 16, G, PC)
    Whp = Wh[_PERM, :]
    return _head(pp[0], pp[1], Whp, bh.reshape(1, H), Wo,
                 bo.reshape(1, OUT))


# speedup vs baseline: 13.1597x; 1.4323x over previous
import functools

import jax
import jax.numpy as jnp
import numpy as np
from jax import lax
from jax.experimental import pallas as pl
from jax.experimental.pallas import tpu as pltpu
from jax.experimental.pallas import tpu_sc as plsc

N = 100000
E = 1600000
C = 32
HC = 16
CONV = 8
GH = 64
H = 64
OUT = 32
G = 64
EPS = 1e-5

RW = 128
NROWP = 12544
RPT = NROWP // 16
R = 8
NWIN = RPT // R
NSP = 6256
NSL = N - 15 * NSP
ZC = 368
CHK = 3128
CHL = NSL - CHK

NP = 100352
BN = 1024
NB = NP // BN
BNP = BN // 8
BNZ = BN * C // RW

PSP = 6256
PLAST = 6160
PCK = 512
PFULL = 12
PT0 = PSP - PFULL * PCK
PT1 = PLAST - PFULL * PCK
LAYERS = CONV + 1
PC = LAYERS * HC


def _segsum_body(h2, srcaug, dst2, aggr, idx_s, idx_d, rows, zbuf, acc,
                 sem_g, sem_s):
    c = lax.axis_index("c")
    s = lax.axis_index("s")
    t0 = s * NSP

    def zloop(i, carry):
        zbuf[i, :] = jnp.zeros((HC,), jnp.float32)
        return carry

    lax.fori_loop(0, ZC, zloop, 0)
    for k in range(16):
        pltpu.sync_copy(zbuf, acc.at[pl.ds(t0 + k * ZC, ZC)])

    @pl.when(s < 15)
    def _z1():
        pltpu.sync_copy(zbuf, acc.at[pl.ds(t0 + 16 * ZC, ZC)])

    @pl.when(s == 15)
    def _z2():
        pltpu.sync_copy(zbuf.at[pl.ds(0, NSL - 16 * ZC)],
                        acc.at[pl.ds(t0 + 16 * ZC, NSL - 16 * ZC)])
        pltpu.sync_copy(zbuf.at[pl.ds(0, 8)], acc.at[pl.ds(N, 8)])

    plsc.subcore_barrier()

    base0 = s * RPT

    def window(w, carry):
        r0 = base0 + w * R
        pltpu.sync_copy(srcaug.at[pl.ds(c * NROWP + r0, R)], idx_s)
        pltpu.sync_copy(dst2.at[pl.ds(r0, R)], idx_d)
        cps = [pltpu.async_copy(h2.at[idx_s.at[j]],
                                rows.at[pl.ds(j * RW, RW)], sem_g)
               for j in range(R)]
        for cp in cps:
            cp.wait()
        cps2 = [pltpu.async_copy(rows.at[pl.ds(j * RW, RW)],
                                 acc.at[idx_d.at[j]], sem_s, add=True)
                for j in range(R)]
        for cp in cps2:
            cp.wait()
        return carry

    lax.fori_loop(0, NWIN, window, 0)

    plsc.subcore_barrier()
    pltpu.sync_copy(acc.at[pl.ds(t0, CHK)], aggr.at[pl.ds(c * NP + t0, CHK)])

    @pl.when(s < 15)
    def _w1():
        pltpu.sync_copy(acc.at[pl.ds(t0 + CHK, CHK)],
                        aggr.at[pl.ds(c * NP + t0 + CHK, CHK)])

    @pl.when(s == 15)
    def _w2():
        pltpu.sync_copy(acc.at[pl.ds(t0 + CHK, CHL)],
                        aggr.at[pl.ds(c * NP + t0 + CHK, CHL)])


_segsum = pl.kernel(
    _segsum_body,
    out_type=jax.ShapeDtypeStruct((2 * NP, HC), jnp.float32),
    mesh=plsc.VectorSubcoreMesh(core_axis_name="c", subcore_axis_name="s"),
    scratch_types=[
        pltpu.VMEM((R, RW), jnp.int32),
        pltpu.VMEM((R, RW), jnp.int32),
        pltpu.VMEM((R * RW, HC), jnp.float32),
        pltpu.VMEM((ZC, HC), jnp.float32),
        pltpu.VMEM_SHARED((N + 8, HC), jnp.float32),
        pltpu.SemaphoreType.DMA,
        pltpu.SemaphoreType.DMA,
    ],
    compiler_params=pltpu.CompilerParams(use_tc_tiling_on_sc=False),
)


def _pool_body(h0, h1, h2, h3, h4, h5, h6, h7, h8, batch, out,
               bbuf, acc, bufs, sem):
    hs = (h0, h1, h2, h3, h4, h5, h6, h7, h8)
    c = lax.axis_index("c")
    s = lax.axis_index("s")
    t0 = s * PSP

    def init_g(g, carry):
        for k in range(LAYERS):
            acc[g, pl.ds(k * HC, HC)] = jnp.full((HC,), -jnp.inf,
                                                 jnp.float32)
        return carry

    lax.fori_loop(0, G, init_g, 0)

    @pl.when(s < 15)
    def _():
        pltpu.sync_copy(batch.at[pl.ds(t0, PSP)], bbuf)

    @pl.when(s == 15)
    def _():
        pltpu.sync_copy(batch.at[pl.ds(t0, PLAST)], bbuf.at[pl.ds(0, PLAST)])

    def do_rows(off, nrows):
        def rloop(r16, carry):
            bvec = bbuf[pl.ds(off + r16 * 16, 16)]
            for l in range(16):
                b = bvec[l]
                r = r16 * 16 + l
                for k in range(LAYERS):
                    sl = acc[b, pl.ds(k * HC, HC)]
                    acc[b, pl.ds(k * HC, HC)] = jnp.maximum(sl, bufs[k][r, :])
            return carry
        lax.fori_loop(0, nrows // 16, rloop, 0)

    def load_chunk(off, nrows):
        cps = [pltpu.async_copy(hs[k].at[pl.ds(c * NP + t0 + off, nrows)],
                                bufs[k].at[pl.ds(0, nrows)], sem)
               for k in range(LAYERS)]
        for cp in cps:
            cp.wait()

    for ch in range(PFULL):
        load_chunk(ch * PCK, PCK)
        do_rows(ch * PCK, PCK)

    @pl.when(s < 15)
    def _tail0():
        load_chunk(PFULL * PCK, PT0)
        do_rows(PFULL * PCK, PT0)

    @pl.when(s == 15)
    def _tail1():
        load_chunk(PFULL * PCK, PT1)
        do_rows(PFULL * PCK, PT1)

    wid = c * 16 + s
    pltpu.sync_copy(acc, out.at[wid])


_pool = pl.kernel(
    _pool_body,
    out_type=jax.ShapeDtypeStruct((32, G, PC), jnp.float32),
    mesh=plsc.VectorSubcoreMesh(core_axis_name="c", subcore_axis_name="s"),
    scratch_types=[
        pltpu.VMEM((PSP,), jnp.int32),
        pltpu.VMEM((G, PC), jnp.float32),
        [pltpu.VMEM((PCK, HC), jnp.float32) for _ in range(LAYERS)],
        pltpu.SemaphoreType.DMA,
    ],
    compiler_params=pltpu.CompilerParams(use_tc_tiling_on_sc=False),
)


def _tile_ch(v):
    return jnp.concatenate(
        [jnp.tile(v[:, :HC], (1, 8)), jnp.tile(v[:, HC:], (1, 8))], axis=1)


def _embed_body(x_ref, embD_ref, out_ref):
    tpat = lax.broadcasted_iota(jnp.int32, (1, 64), 1) & 7
    oh = (x_ref[0] == tpat).astype(jnp.float32)
    hp = jnp.dot(oh, embD_ref[...],
                 preferred_element_type=jnp.float32)
    out_ref[0] = hp[:, :RW]
    out_ref[1] = hp[:, RW:]


def _node_mask(i):
    prow = lax.broadcasted_iota(jnp.int32, (BNP, 2 * RW), 0)
    klane = (lax.broadcasted_iota(jnp.int32, (BNP, 2 * RW), 1) // HC) & 7
    node = 8 * (i * BNP + prow) + klane
    return (node < N).astype(jnp.float32)


def _passA_body(h_ref, a_ref, W1D_ref, b1D_ref, W2D_ref, b2D_ref,
                z2_ref, st_ref):
    z = jnp.concatenate([h_ref[0] + a_ref[0], h_ref[1] + a_ref[1]],
                        axis=1)
    z1 = jnp.maximum(
        jnp.dot(z, W1D_ref[...], preferred_element_type=jnp.float32)
        + b1D_ref[...], 0.0)
    z2 = (jnp.dot(z1, W2D_ref[...], preferred_element_type=jnp.float32)
          + b2D_ref[...])
    z2_ref[...] = z2
    z2m = z2 * _node_mask(pl.program_id(0))
    s1 = jnp.sum(z2m, axis=0, keepdims=True)
    s2 = jnp.sum(z2m * z2, axis=0, keepdims=True)
    fold = lambda s, h: sum(
        s[:, h * RW + k * HC:h * RW + (k + 1) * HC] for k in range(8))
    st_ref[0] = jnp.concatenate(
        [fold(s1, 0), fold(s1, 1), fold(s2, 0), fold(s2, 1)], axis=1)


def _passB_body(st_ref, z2_ref, h_ref, out_ref):
    sums = jnp.sum(st_ref[...], axis=(0, 1)).reshape(1, 2 * C)
    mean = sums[:, :C] * (1.0 / N)
    var = sums[:, C:] * (1.0 / N) - mean * mean
    scale = lax.rsqrt(var + EPS)
    meanD = _tile_ch(mean)
    scaleD = _tile_ch(scale)
    zn = (z2_ref[...] - meanD) * scaleD
    out_ref[0] = jnp.maximum(h_ref[0] + zn[:, :RW], 0.0)
    out_ref[1] = jnp.maximum(h_ref[1] + zn[:, RW:], 0.0)


def _head_body(p0_ref, p1_ref, Wh_ref, bh_ref, Wo_ref, bo_ref, out_ref):
    m0 = jnp.max(p0_ref[...], axis=0)
    m1 = jnp.max(p1_ref[...], axis=0)
    pooled = jnp.concatenate([m0, m1], axis=1)
    pooled = jnp.where(jnp.isfinite(pooled), pooled, 0.0)
    hh = jnp.maximum(
        jnp.dot(pooled, Wh_ref[...], preferred_element_type=jnp.float32)
        + bh_ref[...], 0.0)
    out_ref[...] = (jnp.dot(hh, Wo_ref[...],
                            preferred_element_type=jnp.float32)
                    + bo_ref[...])


_embed = pl.pallas_call(
    _embed_body,
    grid=(NB,),
    in_specs=[
        pl.BlockSpec((1, BNP, 64), lambda i: (i, 0, 0)),
        pl.BlockSpec((64, 2 * RW), lambda i: (0, 0)),
    ],
    out_specs=pl.BlockSpec((2, BNP, RW), lambda i: (0, i, 0)),
    out_shape=jax.ShapeDtypeStruct((2, NP // 8, RW), jnp.float32),
)

_passA = pl.pallas_call(
    _passA_body,
    grid=(NB,),
    in_specs=[
        pl.BlockSpec((2, BNP, RW), lambda i: (0, i, 0)),
        pl.BlockSpec((2, BNP, RW), lambda i: (0, i, 0)),
        pl.BlockSpec((2 * RW, 4 * RW), lambda i: (0, 0)),
        pl.BlockSpec((1, 4 * RW), lambda i: (0, 0)),
        pl.BlockSpec((4 * RW, 2 * RW), lambda i: (0, 0)),
        pl.BlockSpec((1, 2 * RW), lambda i: (0, 0)),
    ],
    out_specs=[
        pl.BlockSpec((BNP, 2 * RW), lambda i: (i, 0)),
        pl.BlockSpec((1, 1, 2 * C), lambda i: (i, 0, 0)),
    ],
    out_shape=[
        jax.ShapeDtypeStruct((NP // 8, 2 * RW), jnp.float32),
        jax.ShapeDtypeStruct((NB, 1, 2 * C), jnp.float32),
    ],
)

_passB = pl.pallas_call(
    _passB_body,
    grid=(NB,),
    in_specs=[
        pl.BlockSpec((NB, 1, 2 * C), lambda i: (0, 0, 0)),
        pl.BlockSpec((BNP, 2 * RW), lambda i: (i, 0)),
        pl.BlockSpec((2, BNP, RW), lambda i: (0, i, 0)),
    ],
    out_specs=pl.BlockSpec((2, BNP, RW), lambda i: (0, i, 0)),
    out_shape=jax.ShapeDtypeStruct((2, NP // 8, RW), jnp.float32),
)

_head = pl.pallas_call(
    _head_body,
    out_shape=jax.ShapeDtypeStruct((G, OUT), jnp.float32),
)

_PERM = np.array([k * C + c * HC + j
                  for c in range(2) for k in range(LAYERS)
                  for j in range(HC)])


def kernel(x, edge_index, batch, emb, W1, b1, W2, b2, Wh, bh, Wo, bo):
    src = edge_index[0]
    dst = edge_index[1]
    npad = NROWP * RW - E
    pad_i = jnp.arange(npad, dtype=jnp.int32)
    srcp = jnp.concatenate([src, (pad_i * 8) % N])
    dstp = jnp.concatenate([dst, N + (pad_i % 8)])
    srcaug = jnp.concatenate([srcp, srcp + NP]).reshape(2 * NROWP, RW)
    dst2 = dstp.reshape(NROWP, RW)
    xpad = jnp.concatenate([x, jnp.zeros((NP - N,), jnp.int32)])
    xD = jnp.repeat(xpad.reshape(NP // 8, 8), 8, axis=1).reshape(NB, BNP, 64)
    embp = jnp.zeros((8, C), jnp.float32).at[:6].set(emb)
    eye8 = jnp.eye(8, dtype=jnp.float32)
    embD = jnp.einsum("kK,tca->ktcKa", eye8,
                      embp.reshape(8, 2, HC)).reshape(64, 2 * RW)
    W1D = [jnp.einsum("cao,kK->ckaKo", W1[i].reshape(2, HC, GH),
                      eye8).reshape(2 * RW, 4 * RW) for i in range(CONV)]
    b1D = [jnp.tile(b1[i], 8).reshape(1, 4 * RW) for i in range(CONV)]
    W2D = [jnp.einsum("uca,kK->kucKa", W2[i].reshape(GH, 2, HC),
                      eye8).reshape(4 * RW, 2 * RW) for i in range(CONV)]
    b2D = [jnp.tile(b2[i].reshape(2, 1, HC),
                    (1, 8, 1)).reshape(1, 2 * RW) for i in range(CONV)]

    h = _embed(xD, embD)
    hs = [h]
    for i in range(CONV):
        aggr2 = _segsum(h.reshape(2 * NP, HC), srcaug, dst2)
        z2, stats = _passA(h, aggr2.reshape(2, NP // 8, RW), W1D[i],
                           b1D[i], W2D[i], b2D[i])
        h = _passB(stats, z2, h)
        hs.append(h)

    pp = _pool(*[hh.reshape(2 * NP, HC) for hh in hs], batch)
    pp = pp.reshape(2, 16, G, PC)
    Whp = Wh[_PERM, :]
    return _head(pp[0], pp[1], Whp, bh.reshape(1, H), Wo,
                 bo.reshape(1, OUT))


# speedup vs baseline: 18.3342x; 1.3932x over previous
import functools

import jax
import jax.numpy as jnp
import numpy as np
from jax import lax
from jax.experimental import pallas as pl
from jax.experimental.pallas import tpu as pltpu
from jax.experimental.pallas import tpu_sc as plsc

N = 100000
E = 1600000
C = 32
HC = 16
CONV = 8
GH = 64
H = 64
OUT = 32
G = 64
EPS = 1e-5

RW = 128
NROWP = 12544
RPT = NROWP // 16
R = 4
WE = R * RW
NWIN = RPT // R
NPAIR = NWIN // 2
NSP = 6256
NSL = N - 15 * NSP
CHK = 3128
CHL = NSL - CHK

NP = 100352
BN = 1024
NB = NP // BN
BNP = BN // 8
BNZ = BN * C // RW

PSP = 6256
PLAST = 6160
PCK = 512
PFULL = 12
PT0 = PSP - PFULL * PCK
PT1 = PLAST - PFULL * PCK
LAYERS = CONV + 1
PC = LAYERS * HC


def _segsum_body(h2, srcaug, dst2, aggr, idx_s2, idx_d2, rows2, acc,
                 sem_i0, sem_i1, sem_g0, sem_g1, sem_s0, sem_s1):
    c = lax.axis_index("c")
    s = lax.axis_index("s")
    t0 = s * NSP
    sem_i = (sem_i0, sem_i1)
    sem_g = (sem_g0, sem_g1)
    sem_s = (sem_s0, sem_s1)

    def zloop(i, carry):
        rows2[0, i, :] = jnp.zeros((HC,), jnp.float32)
        return carry

    lax.fori_loop(0, WE, zloop, 0)
    for k in range(12):
        pltpu.sync_copy(rows2.at[0], acc.at[pl.ds(t0 + k * WE, WE)])

    @pl.when(s < 15)
    def _z1():
        pltpu.sync_copy(rows2.at[0].at[pl.ds(0, NSP - 12 * WE)],
                        acc.at[pl.ds(t0 + 12 * WE, NSP - 12 * WE)])

    @pl.when(s == 15)
    def _z2():
        pltpu.sync_copy(rows2.at[0].at[pl.ds(0, NSL - 12 * WE)],
                        acc.at[pl.ds(t0 + 12 * WE, NSL - 12 * WE)])
        pltpu.sync_copy(rows2.at[0].at[pl.ds(0, 8)], acc.at[pl.ds(N, 8)])

    plsc.subcore_barrier()

    base = s * RPT

    def fire_idx(w, b):
        pltpu.async_copy(
            srcaug.at[pl.ds((c * NROWP + base + w * R) * RW, WE)],
            idx_s2.at[b], sem_i[b])
        pltpu.async_copy(dst2.at[pl.ds(base + w * R, R)], idx_d2.at[b],
                         sem_i[b])

    def wait_idx(w, b):
        pltpu.make_async_copy(
            srcaug.at[pl.ds((c * NROWP + base + w * R) * RW, WE)],
            idx_s2.at[b], sem_i[b]).wait()
        pltpu.make_async_copy(dst2.at[pl.ds(base + w * R, R)],
                              idx_d2.at[b], sem_i[b]).wait()

    def fire_g(b):
        pltpu.async_copy(h2.at[idx_s2.at[b]], rows2.at[b], sem_g[b])

    def wait_g(b):
        pltpu.make_async_copy(h2.at[idx_s2.at[b]], rows2.at[b],
                              sem_g[b]).wait()

    def fire_s(b):
        for j in range(R):
            pltpu.async_copy(rows2.at[b].at[pl.ds(j * RW, RW)],
                             acc.at[idx_d2.at[b].at[j]], sem_s[b],
                             add=True)

    def wait_s(b):
        for j in range(R):
            pltpu.make_async_copy(rows2.at[b].at[pl.ds(j * RW, RW)],
                                  acc.at[idx_d2.at[b].at[j]],
                                  sem_s[b]).wait()

    fire_idx(0, 0)

    def pair(k, carry):
        w0 = 2 * k
        w1 = 2 * k + 1
        wait_idx(w0, 0)

        @pl.when(k > 0)
        def _():
            wait_s(0)

        fire_g(0)

        @pl.when(k > 0)
        def _():
            wait_g(1)
            fire_s(1)

        fire_idx(w1, 1)
        wait_idx(w1, 1)

        @pl.when(k > 0)
        def _():
            wait_s(1)

        fire_g(1)
        wait_g(0)
        fire_s(0)

        @pl.when(k < NPAIR - 1)
        def _():
            fire_idx(w1 + 1, 0)

        return carry

    lax.fori_loop(0, NPAIR, pair, 0)
    wait_g(1)
    fire_s(1)
    wait_s(0)
    wait_s(1)

    plsc.subcore_barrier()
    pltpu.sync_copy(acc.at[pl.ds(t0, CHK)], aggr.at[pl.ds(c * NP + t0, CHK)])

    @pl.when(s < 15)
    def _w1():
        pltpu.sync_copy(acc.at[pl.ds(t0 + CHK, CHK)],
                        aggr.at[pl.ds(c * NP + t0 + CHK, CHK)])

    @pl.when(s == 15)
    def _w2():
        pltpu.sync_copy(acc.at[pl.ds(t0 + CHK, CHL)],
                        aggr.at[pl.ds(c * NP + t0 + CHK, CHL)])


_segsum = pl.kernel(
    _segsum_body,
    out_type=jax.ShapeDtypeStruct((2 * NP, HC), jnp.float32),
    mesh=plsc.VectorSubcoreMesh(core_axis_name="c", subcore_axis_name="s"),
    scratch_types=[
        pltpu.VMEM((2, WE), jnp.int32),
        pltpu.VMEM((2, R, RW), jnp.int32),
        pltpu.VMEM((2, WE, HC), jnp.float32),
        pltpu.VMEM_SHARED((N + 8, HC), jnp.float32),
        pltpu.SemaphoreType.DMA,
        pltpu.SemaphoreType.DMA,
        pltpu.SemaphoreType.DMA,
        pltpu.SemaphoreType.DMA,
        pltpu.SemaphoreType.DMA,
        pltpu.SemaphoreType.DMA,
    ],
    compiler_params=pltpu.CompilerParams(use_tc_tiling_on_sc=False),
)


def _pool_body(h0, h1, h2, h3, h4, h5, h6, h7, h8, batch, out,
               bbuf, acc, bufs, sem):
    hs = (h0, h1, h2, h3, h4, h5, h6, h7, h8)
    c = lax.axis_index("c")
    s = lax.axis_index("s")
    t0 = s * PSP

    def init_g(g, carry):
        for k in range(LAYERS):
            acc[g, pl.ds(k * HC, HC)] = jnp.full((HC,), -jnp.inf,
                                                 jnp.float32)
        return carry

    lax.fori_loop(0, G, init_g, 0)

    @pl.when(s < 15)
    def _():
        pltpu.sync_copy(batch.at[pl.ds(t0, PSP)], bbuf)

    @pl.when(s == 15)
    def _():
        pltpu.sync_copy(batch.at[pl.ds(t0, PLAST)], bbuf.at[pl.ds(0, PLAST)])

    def do_rows(off, nrows):
        def rloop(r16, carry):
            bvec = bbuf[pl.ds(off + r16 * 16, 16)]
            for l in range(16):
                b = bvec[l]
                r = r16 * 16 + l
                for k in range(LAYERS):
                    sl = acc[b, pl.ds(k * HC, HC)]
                    acc[b, pl.ds(k * HC, HC)] = jnp.maximum(sl, bufs[k][r, :])
            return carry
        lax.fori_loop(0, nrows // 16, rloop, 0)

    def load_chunk(off, nrows):
        cps = [pltpu.async_copy(hs[k].at[pl.ds(c * NP + t0 + off, nrows)],
                                bufs[k].at[pl.ds(0, nrows)], sem)
               for k in range(LAYERS)]
        for cp in cps:
            cp.wait()

    for ch in range(PFULL):
        load_chunk(ch * PCK, PCK)
        do_rows(ch * PCK, PCK)

    @pl.when(s < 15)
    def _tail0():
        load_chunk(PFULL * PCK, PT0)
        do_rows(PFULL * PCK, PT0)

    @pl.when(s == 15)
    def _tail1():
        load_chunk(PFULL * PCK, PT1)
        do_rows(PFULL * PCK, PT1)

    wid = c * 16 + s
    pltpu.sync_copy(acc, out.at[wid])


_pool = pl.kernel(
    _pool_body,
    out_type=jax.ShapeDtypeStruct((32, G, PC), jnp.float32),
    mesh=plsc.VectorSubcoreMesh(core_axis_name="c", subcore_axis_name="s"),
    scratch_types=[
        pltpu.VMEM((PSP,), jnp.int32),
        pltpu.VMEM((G, PC), jnp.float32),
        [pltpu.VMEM((PCK, HC), jnp.float32) for _ in range(LAYERS)],
        pltpu.SemaphoreType.DMA,
    ],
    compiler_params=pltpu.CompilerParams(use_tc_tiling_on_sc=False),
)


def _tile_ch(v):
    return jnp.concatenate(
        [jnp.tile(v[:, :HC], (1, 8)), jnp.tile(v[:, HC:], (1, 8))], axis=1)


def _embed_body(x_ref, embD_ref, out_ref):
    tpat = lax.broadcasted_iota(jnp.int32, (1, 64), 1) & 7
    oh = (x_ref[0] == tpat).astype(jnp.float32)
    hp = jnp.dot(oh, embD_ref[...],
                 preferred_element_type=jnp.float32)
    out_ref[0] = hp[:, :RW]
    out_ref[1] = hp[:, RW:]


def _node_mask(i):
    prow = lax.broadcasted_iota(jnp.int32, (BNP, 2 * RW), 0)
    klane = (lax.broadcasted_iota(jnp.int32, (BNP, 2 * RW), 1) // HC) & 7
    node = 8 * (i * BNP + prow) + klane
    return node < N


def _passA_body(h_ref, a_ref, W1D_ref, b1D_ref, W2D_ref, b2D_ref,
                z2_ref, st_ref):
    z = jnp.concatenate([h_ref[0] + a_ref[0], h_ref[1] + a_ref[1]],
                        axis=1)
    z1 = jnp.maximum(
        jnp.dot(z, W1D_ref[...], preferred_element_type=jnp.float32)
        + b1D_ref[...], 0.0)
    z2 = (jnp.dot(z1, W2D_ref[...], preferred_element_type=jnp.float32)
          + b2D_ref[...])
    z2_ref[...] = z2
    z2m = jnp.where(_node_mask(pl.program_id(0)), z2, 0.0)
    s1 = jnp.sum(z2m, axis=0, keepdims=True)
    s2 = jnp.sum(z2m * z2m, axis=0, keepdims=True)
    fold = lambda s, h: sum(
        s[:, h * RW + k * HC:h * RW + (k + 1) * HC] for k in range(8))
    st_ref[0] = jnp.concatenate(
        [fold(s1, 0), fold(s1, 1), fold(s2, 0), fold(s2, 1)], axis=1)


def _passB_body(st_ref, z2_ref, h_ref, out_ref):
    sums = jnp.sum(st_ref[...], axis=(0, 1)).reshape(1, 2 * C)
    mean = sums[:, :C] * (1.0 / N)
    var = sums[:, C:] * (1.0 / N) - mean * mean
    scale = lax.rsqrt(var + EPS)
    meanD = _tile_ch(mean)
    scaleD = _tile_ch(scale)
    zn = (z2_ref[...] - meanD) * scaleD
    out_ref[0] = jnp.maximum(h_ref[0] + zn[:, :RW], 0.0)
    out_ref[1] = jnp.maximum(h_ref[1] + zn[:, RW:], 0.0)


def _head_body(p0_ref, p1_ref, Wh_ref, bh_ref, Wo_ref, bo_ref, out_ref):
    m0 = jnp.max(p0_ref[...], axis=0)
    m1 = jnp.max(p1_ref[...], axis=0)
    pooled = jnp.concatenate([m0, m1], axis=1)
    pooled = jnp.where(jnp.isfinite(pooled), pooled, 0.0)
    hh = jnp.maximum(
        jnp.dot(pooled, Wh_ref[...], preferred_element_type=jnp.float32)
        + bh_ref[...], 0.0)
    out_ref[...] = (jnp.dot(hh, Wo_ref[...],
                            preferred_element_type=jnp.float32)
                    + bo_ref[...])


_embed = pl.pallas_call(
    _embed_body,
    grid=(NB,),
    in_specs=[
        pl.BlockSpec((1, BNP, 64), lambda i: (i, 0, 0)),
        pl.BlockSpec((64, 2 * RW), lambda i: (0, 0)),
    ],
    out_specs=pl.BlockSpec((2, BNP, RW), lambda i: (0, i, 0)),
    out_shape=jax.ShapeDtypeStruct((2, NP // 8, RW), jnp.float32),
)

_passA = pl.pallas_call(
    _passA_body,
    grid=(NB,),
    in_specs=[
        pl.BlockSpec((2, BNP, RW), lambda i: (0, i, 0)),
        pl.BlockSpec((2, BNP, RW), lambda i: (0, i, 0)),
        pl.BlockSpec((2 * RW, 4 * RW), lambda i: (0, 0)),
        pl.BlockSpec((1, 4 * RW), lambda i: (0, 0)),
        pl.BlockSpec((4 * RW, 2 * RW), lambda i: (0, 0)),
        pl.BlockSpec((1, 2 * RW), lambda i: (0, 0)),
    ],
    out_specs=[
        pl.BlockSpec((BNP, 2 * RW), lambda i: (i, 0)),
        pl.BlockSpec((1, 1, 2 * C), lambda i: (i, 0, 0)),
    ],
    out_shape=[
        jax.ShapeDtypeStruct((NP // 8, 2 * RW), jnp.float32),
        jax.ShapeDtypeStruct((NB, 1, 2 * C), jnp.float32),
    ],
)

_passB = pl.pallas_call(
    _passB_body,
    grid=(NB,),
    in_specs=[
        pl.BlockSpec((NB, 1, 2 * C), lambda i: (0, 0, 0)),
        pl.BlockSpec((BNP, 2 * RW), lambda i: (i, 0)),
        pl.BlockSpec((2, BNP, RW), lambda i: (0, i, 0)),
    ],
    out_specs=pl.BlockSpec((2, BNP, RW), lambda i: (0, i, 0)),
    out_shape=jax.ShapeDtypeStruct((2, NP // 8, RW), jnp.float32),
)

_head = pl.pallas_call(
    _head_body,
    out_shape=jax.ShapeDtypeStruct((G, OUT), jnp.float32),
)

_PERM = np.array([k * C + c * HC + j
                  for c in range(2) for k in range(LAYERS)
                  for j in range(HC)])


def kernel(x, edge_index, batch, emb, W1, b1, W2, b2, Wh, bh, Wo, bo):
    src = edge_index[0]
    dst = edge_index[1]
    npad = NROWP * RW - E
    pad_i = jnp.arange(npad, dtype=jnp.int32)
    srcp = jnp.concatenate([src, (pad_i * 8) % N])
    dstp = jnp.concatenate([dst, N + (pad_i % 8)])
    srcaug = jnp.concatenate([srcp, srcp + NP])
    dst2 = dstp.reshape(NROWP, RW)
    xpad = jnp.concatenate([x, jnp.zeros((NP - N,), jnp.int32)])
    xD = jnp.repeat(xpad.reshape(NP // 8, 8), 8, axis=1).reshape(NB, BNP, 64)
    embp = jnp.zeros((8, C), jnp.float32).at[:6].set(emb)
    eye8 = jnp.eye(8, dtype=jnp.float32)
    embD = jnp.einsum("kK,tca->ktcKa", eye8,
                      embp.reshape(8, 2, HC)).reshape(64, 2 * RW)
    W1D = [jnp.einsum("cao,kK->ckaKo", W1[i].reshape(2, HC, GH),
                      eye8).reshape(2 * RW, 4 * RW) for i in range(CONV)]
    b1D = [jnp.tile(b1[i], 8).reshape(1, 4 * RW) for i in range(CONV)]
    W2D = [jnp.einsum("uca,kK->kucKa", W2[i].reshape(GH, 2, HC),
                      eye8).reshape(4 * RW, 2 * RW) for i in range(CONV)]
    b2D = [jnp.tile(b2[i].reshape(2, 1, HC),
                    (1, 8, 1)).reshape(1, 2 * RW) for i in range(CONV)]

    h = _embed(xD, embD)
    hs = [h]
    for i in range(CONV):
        aggr2 = _segsum(h.reshape(2 * NP, HC), srcaug, dst2)
        z2, stats = _passA(h, aggr2.reshape(2, NP // 8, RW), W1D[i],
                           b1D[i], W2D[i], b2D[i])
        h = _passB(stats, z2, h)
        hs.append(h)

    pp = _pool(*[hh.reshape(2 * NP, HC) for hh in hs], batch)
    pp = pp.reshape(2, 16, G, PC)
    Whp = Wh[_PERM, :]
    return _head(pp[0], pp[1], Whp, bh.reshape(1, H), Wo,
                 bo.reshape(1, OUT))
